# baseline (speedup 1.0000x reference)
import functools
import jax
import jax.numpy as jnp
from jax import lax
from jax.experimental import pallas as pl
from jax.experimental.pallas import tpu as pltpu
from jax.experimental.pallas import tpu_sc as plsc

N_NODES = 10000
N_EDGES = 320000
IN_CH = 128
HID = 64

NC = 2
NS = 16
NW = NC * NS
K = 128
CHUNKS = 80
EPW = K * CHUNKS
E_PAD = NW * EPW
NPAD = 10240
RPT = NPAD // NS
NPT = N_NODES // NS
JUNK = NPAD - 1

NBUF = 8
GAH = 4
DBUF = 4

NBLK = 1000
EBLK = 2048

_mesh = plsc.VectorSubcoreMesh(core_axis_name="c", subcore_axis_name="s")


@functools.partial(
    pl.kernel,
    out_type=jax.ShapeDtypeStruct((NC, N_NODES, 16), jnp.float32),
    mesh=_mesh,
    compiler_params=pltpu.CompilerParams(use_tc_tiling_on_sc=False),
    scratch_types=[
        pltpu.VMEM((CHUNKS, K), jnp.int32),
        pltpu.VMEM((K, 16), jnp.float32),
        pltpu.VMEM_SHARED((NPAD, 16), jnp.float32),
        pltpu.SemaphoreType.DMA,
    ],
)
def _sc_degree(dst_hbm, zeros_hbm, ones_hbm, out_hbm, idx_v, ones_v, acc_sh, sem):
    cid = lax.axis_index("c")
    sid = lax.axis_index("s")
    wid = sid * NC + cid
    r0 = sid * RPT
    pltpu.sync_copy(zeros_hbm.at[pl.ds(r0, RPT)], acc_sh.at[pl.ds(r0, RPT)])
    pltpu.sync_copy(ones_hbm, ones_v)
    pltpu.sync_copy(dst_hbm.at[wid], idx_v)
    plsc.subcore_barrier()

    def body(c, carry):
        pltpu.sync_copy(ones_v, acc_sh.at[idx_v.at[c]], add=True)
        return carry

    lax.fori_loop(0, CHUNKS, body, 0)
    plsc.subcore_barrier()
    n0 = sid * NPT
    pltpu.sync_copy(acc_sh.at[pl.ds(n0, NPT)], out_hbm.at[cid, pl.ds(n0, NPT)])


@functools.partial(
    pl.kernel,
    out_type=jax.ShapeDtypeStruct((NC, N_NODES, HID), jnp.float32),
    mesh=_mesh,
    compiler_params=pltpu.CompilerParams(use_tc_tiling_on_sc=False),
    scratch_types=[
        pltpu.VMEM((CHUNKS, K), jnp.int32),
        pltpu.VMEM((CHUNKS, K), jnp.int32),
        [pltpu.VMEM((K, HID), jnp.float32)] * NBUF,
        pltpu.VMEM_SHARED((NPAD, HID), jnp.float32),
        [pltpu.SemaphoreType.DMA] * NBUF,
        [pltpu.SemaphoreType.DMA] * NBUF,
    ],
)
def _sc_aggregate(table_hbm, src_hbm, dst_hbm, zeros_hbm, out_hbm,
                  sidx_v, didx_v, bufs, acc_sh, gsems, ssems):
    cid = lax.axis_index("c")
    sid = lax.axis_index("s")
    wid = sid * NC + cid
    r0 = sid * RPT
    n0 = sid * NPT
    pltpu.sync_copy(zeros_hbm.at[pl.ds(r0, RPT)], acc_sh.at[pl.ds(r0, RPT)])
    pltpu.sync_copy(src_hbm.at[wid], sidx_v)
    pltpu.sync_copy(dst_hbm.at[wid], didx_v)
    plsc.subcore_barrier()

    for c in range(GAH):
        pltpu.async_copy(table_hbm.at[sidx_v.at[c]], bufs[c], gsems[c])

    def slot(c, b):
        pltpu.make_async_copy(table_hbm, bufs[b], gsems[b]).wait()
        pltpu.async_copy(bufs[b], acc_sh.at[didx_v.at[c]], ssems[b], add=True)
        b2 = (b + GAH) % NBUF

        @pl.when(c >= GAH)
        def _():
            pltpu.make_async_copy(bufs[b2], acc_sh.at[didx_v.at[0]], ssems[b2]).wait()

        @pl.when(c + GAH < CHUNKS)
        def _():
            pltpu.async_copy(table_hbm.at[sidx_v.at[c + GAH]], bufs[b2], gsems[b2])

    def body(i, carry):
        for b in range(NBUF):
            slot(i * NBUF + b, b)
        return carry

    lax.fori_loop(0, CHUNKS // NBUF, body, 0)
    for c in range(CHUNKS - GAH, CHUNKS):
        b = c % NBUF
        pltpu.make_async_copy(bufs[b], acc_sh.at[didx_v.at[0]], ssems[b]).wait()
    plsc.subcore_barrier()
    pltpu.sync_copy(acc_sh.at[pl.ds(n0, NPT)], out_hbm.at[cid, pl.ds(n0, NPT)])


@functools.partial(
    pl.kernel,
    out_type=(
        jax.ShapeDtypeStruct((E_PAD, HID), jnp.float32),
        jax.ShapeDtypeStruct((E_PAD, HID), jnp.float32),
    ),
    mesh=_mesh,
    compiler_params=pltpu.CompilerParams(use_tc_tiling_on_sc=False),
    scratch_types=[
        pltpu.VMEM((CHUNKS, K), jnp.int32),
        pltpu.VMEM((CHUNKS, K), jnp.int32),
        [pltpu.VMEM((K, HID), jnp.float32)] * DBUF,
        [pltpu.VMEM((K, HID), jnp.float32)] * DBUF,
        [pltpu.SemaphoreType.DMA] * DBUF,
        [pltpu.SemaphoreType.DMA] * DBUF,
        [pltpu.SemaphoreType.DMA] * DBUF,
        [pltpu.SemaphoreType.DMA] * DBUF,
    ],
)
def _sc_edge_gather(ta_hbm, tb_hbm, fsrc_hbm, fdst_hbm, outa_hbm, outb_hbm,
                    sidx_v, didx_v, bufa, bufb,
                    gsa, gsb, wsa, wsb):
    cid = lax.axis_index("c")
    sid = lax.axis_index("s")
    wid = sid * NC + cid
    base = wid * EPW
    pltpu.sync_copy(fsrc_hbm.at[wid], sidx_v)
    pltpu.sync_copy(fdst_hbm.at[wid], didx_v)

    LAH = 2
    for c in range(LAH):
        pltpu.async_copy(ta_hbm.at[sidx_v.at[c]], bufa[c], gsa[c])
        pltpu.async_copy(tb_hbm.at[didx_v.at[c]], bufb[c], gsb[c])

    def slot(c, b):
        pltpu.make_async_copy(ta_hbm, bufa[b], gsa[b]).wait()
        pltpu.make_async_copy(tb_hbm, bufb[b], gsb[b]).wait()
        pltpu.async_copy(bufa[b], outa_hbm.at[pl.ds(base + c * K, K)], wsa[b])
        pltpu.async_copy(bufb[b], outb_hbm.at[pl.ds(base + c * K, K)], wsb[b])
        b2 = (b + LAH) % DBUF

        @pl.when(c >= DBUF - LAH)
        def _():
            pltpu.make_async_copy(bufa[b2], outa_hbm.at[pl.ds(base, K)], wsa[b2]).wait()
            pltpu.make_async_copy(bufb[b2], outb_hbm.at[pl.ds(base, K)], wsb[b2]).wait()

        @pl.when(c + LAH < CHUNKS)
        def _():
            pltpu.async_copy(ta_hbm.at[sidx_v.at[c + LAH]], bufa[b2], gsa[b2])
            pltpu.async_copy(tb_hbm.at[didx_v.at[c + LAH]], bufb[b2], gsb[b2])

    def body(i, carry):
        for b in range(DBUF):
            slot(i * DBUF + b, b)
        return carry

    lax.fori_loop(0, CHUNKS // DBUF, body, 0)
    for c in range(CHUNKS - LAH, CHUNKS):
        b = c % DBUF
        pltpu.make_async_copy(bufa[b], outa_hbm.at[pl.ds(base, K)], wsa[b]).wait()
        pltpu.make_async_copy(bufb[b], outb_hbm.at[pl.ds(base, K)], wsb[b]).wait()


def _silu(v):
    return v / (1.0 + jnp.exp(-v))


def _embed_body(x_ref, ts_ref, degp_ref, W_in_ref, b_in_ref, W_t1_ref, b_t1_ref,
                W_t2_ref, b_t2_ref, W_c1_ref, h0_ref, g1s_ref, dinv_ref):
    x = x_ref[...]
    h = jnp.dot(x, W_in_ref[...], preferred_element_type=jnp.float32) + b_in_ref[...]
    t = ts_ref[...].astype(jnp.float32)
    half = HID // 2
    ramp = lax.broadcasted_iota(jnp.int32, (1, half), 1).astype(jnp.float32)
    freqs = jnp.exp(-jnp.log(10000.0) * ramp / (half - 1))
    args = t * freqs
    emb = jnp.concatenate([jnp.sin(args), jnp.cos(args)], axis=-1)
    te = jnp.dot(emb, W_t1_ref[...], preferred_element_type=jnp.float32) + b_t1_ref[...]
    te = _silu(te)
    te = jnp.dot(te, W_t2_ref[...], preferred_element_type=jnp.float32) + b_t2_ref[...]
    h0 = h + te
    deg = degp_ref[0, :, 0:1] + degp_ref[1, :, 0:1] + 1.0
    dinv = lax.rsqrt(deg)
    g1 = jnp.dot(h0, W_c1_ref[...], preferred_element_type=jnp.float32)
    h0_ref[...] = h0
    g1s_ref[...] = g1 * dinv
    dinv_ref[...] = dinv


def _layer_body(aggp_ref, gs_ref, dinv_ref, b_ref, W_next_ref, out_ref):
    dinv = dinv_ref[...]
    z = (aggp_ref[0] + aggp_ref[1] + gs_ref[...]) * dinv + b_ref[...]
    z = jnp.maximum(z, 0.0)
    out_ref[...] = jnp.dot(z, W_next_ref[...], preferred_element_type=jnp.float32) * dinv


def _final_body(aggp_ref, gs_ref, dinv_ref, b_ref, Wfa_ref, bfa_ref, Wfb_ref,
                h1_ref, ta_ref, tb_ref):
    dinv = dinv_ref[...]
    h1 = (aggp_ref[0] + aggp_ref[1] + gs_ref[...]) * dinv + b_ref[...]
    h1_ref[...] = h1
    ta_ref[...] = jnp.dot(h1, Wfa_ref[...], preferred_element_type=jnp.float32) + bfa_ref[...]
    tb_ref[...] = jnp.dot(h1, Wfb_ref[...], preferred_element_type=jnp.float32)


def _decoder_body(sa_ref, sb_ref, wf2_ref, bf2_ref, out_ref):
    s = _silu(sa_ref[...] + sb_ref[...])
    out_ref[...] = jnp.dot(s, wf2_ref[...], preferred_element_type=jnp.float32) + bf2_ref[...]


def _full(shape):
    return pl.BlockSpec(shape, lambda i: (0,) * len(shape))


def _rows(blk, width):
    return pl.BlockSpec((blk, width), lambda i: (i, 0))


def kernel(x, edge_index, full_edge_index, time_steps,
           W_in, b_in, W_t1, b_t1, W_t2, b_t2,
           W_c1, b_c1, W_c2, b_c2, W_f1, b_f1, W_f2, b_f2):
    f32 = jnp.float32

    def pad_idx(a, fill):
        a = a.astype(jnp.int32)
        a = jnp.concatenate([a, jnp.full((E_PAD - N_EDGES,), fill, jnp.int32)])
        return a.reshape(NW, CHUNKS, K)

    src_p = pad_idx(edge_index[0], 0)
    dst_p = pad_idx(edge_index[1], JUNK)
    fsrc_p = pad_idx(full_edge_index[0], 0)
    fdst_p = pad_idx(full_edge_index[1], 0)

    ts_c = time_steps.astype(jnp.int32).reshape(N_NODES, 1)

    zeros16 = jnp.zeros((NPAD, 16), f32)
    ones16 = jnp.ones((K, 16), f32)
    zeros64 = jnp.zeros((NPAD, HID), f32)

    degp = _sc_degree(dst_p, zeros16, ones16)

    grid_n = N_NODES // NBLK
    h0, g1s, dinv = pl.pallas_call(
        _embed_body,
        grid=(grid_n,),
        in_specs=[
            _rows(NBLK, IN_CH),
            _rows(NBLK, 1),
            pl.BlockSpec((NC, NBLK, 16), lambda i: (0, i, 0)),
            _full((IN_CH, HID)), _full((1, HID)),
            _full((HID, 4 * HID)), _full((1, 4 * HID)),
            _full((4 * HID, HID)), _full((1, HID)),
            _full((HID, HID)),
        ],
        out_specs=[_rows(NBLK, HID), _rows(NBLK, HID), _rows(NBLK, 1)],
        out_shape=[
            jax.ShapeDtypeStruct((N_NODES, HID), f32),
            jax.ShapeDtypeStruct((N_NODES, HID), f32),
            jax.ShapeDtypeStruct((N_NODES, 1), f32),
        ],
    )(x, ts_c, degp, W_in, b_in.reshape(1, HID),
      W_t1, b_t1.reshape(1, 4 * HID), W_t2, b_t2.reshape(1, HID), W_c1)

    agg1 = _sc_aggregate(g1s, src_p, dst_p, zeros64)

    g2s = pl.pallas_call(
        _layer_body,
        grid=(grid_n,),
        in_specs=[
            pl.BlockSpec((NC, NBLK, HID), lambda i: (0, i, 0)),
            _rows(NBLK, HID), _rows(NBLK, 1), _full((1, HID)), _full((HID, HID)),
        ],
        out_specs=_rows(NBLK, HID),
        out_shape=jax.ShapeDtypeStruct((N_NODES, HID), f32),
    )(agg1, g1s, dinv, b_c1.reshape(1, HID), W_c2)

    agg2 = _sc_aggregate(g2s, src_p, dst_p, zeros64)

    h1, ta, tb = pl.pallas_call(
        _final_body,
        grid=(grid_n,),
        in_specs=[
            pl.BlockSpec((NC, NBLK, HID), lambda i: (0, i, 0)),
            _rows(NBLK, HID), _rows(NBLK, 1), _full((1, HID)),
            _full((HID, HID)), _full((1, HID)), _full((HID, HID)),
        ],
        out_specs=[_rows(NBLK, HID), _rows(NBLK, HID), _rows(NBLK, HID)],
        out_shape=[
            jax.ShapeDtypeStruct((N_NODES, HID), f32),
            jax.ShapeDtypeStruct((N_NODES, HID), f32),
            jax.ShapeDtypeStruct((N_NODES, HID), f32),
        ],
    )(agg2, g2s, dinv, b_c2.reshape(1, HID),
      W_f1[:HID], b_f1.reshape(1, HID), W_f1[HID:])

    sa, sb = _sc_edge_gather(ta, tb, fsrc_p, fdst_p)

    grid_e = E_PAD // EBLK
    logits = pl.pallas_call(
        _decoder_body,
        grid=(grid_e,),
        in_specs=[
            _rows(EBLK, HID), _rows(EBLK, HID),
            _full((HID, 1)), _full((1, 1)),
        ],
        out_specs=_rows(EBLK, 1),
        out_shape=jax.ShapeDtypeStruct((E_PAD, 1), f32),
    )(sa, sb, W_f2, b_f2.reshape(1, 1))

    return (logits[:N_EDGES], h0, h1)


# speedup vs baseline: 1.3965x; 1.3965x over previous
import functools
import jax
import jax.numpy as jnp
from jax import lax
from jax.experimental import pallas as pl
from jax.experimental.pallas import tpu as pltpu
from jax.experimental.pallas import tpu_sc as plsc

N_NODES = 10000
N_EDGES = 320000
IN_CH = 128
HID = 64

NC = 2
NS = 16
NW = NC * NS
K = 128
CHUNKS = 80
EPW = K * CHUNKS
E_PAD = NW * EPW
NPAD = 10240
RPT = NPAD // NS
NPT = N_NODES // NS
JUNK = NPAD - 1

NBUF = 8
GAH = 4
DBUF = 4

NBLK = 1000
EBLK = 2048

_mesh = plsc.VectorSubcoreMesh(core_axis_name="c", subcore_axis_name="s")


@functools.partial(
    pl.kernel,
    out_type=jax.ShapeDtypeStruct((NC, N_NODES, 16), jnp.float32),
    mesh=_mesh,
    compiler_params=pltpu.CompilerParams(use_tc_tiling_on_sc=False),
    scratch_types=[
        pltpu.VMEM((CHUNKS, K), jnp.int32),
        pltpu.VMEM((K, 16), jnp.float32),
        pltpu.VMEM_SHARED((NPAD, 16), jnp.float32),
        pltpu.SemaphoreType.DMA,
    ],
)
def _sc_degree(dst_hbm, zeros_hbm, ones_hbm, out_hbm, idx_v, ones_v, acc_sh, sem):
    cid = lax.axis_index("c")
    sid = lax.axis_index("s")
    wid = sid * NC + cid
    r0 = sid * RPT
    pltpu.sync_copy(zeros_hbm.at[pl.ds(r0, RPT)], acc_sh.at[pl.ds(r0, RPT)])
    pltpu.sync_copy(ones_hbm, ones_v)
    pltpu.sync_copy(dst_hbm.at[wid], idx_v)
    plsc.subcore_barrier()

    def body(c, carry):
        pltpu.sync_copy(ones_v, acc_sh.at[idx_v.at[c]], add=True)
        return carry

    lax.fori_loop(0, CHUNKS, body, 0)
    plsc.subcore_barrier()
    n0 = sid * NPT
    pltpu.sync_copy(acc_sh.at[pl.ds(n0, NPT)], out_hbm.at[cid, pl.ds(n0, NPT)])


@functools.partial(
    pl.kernel,
    out_type=jax.ShapeDtypeStruct((NC, N_NODES, HID), jnp.float32),
    mesh=_mesh,
    compiler_params=pltpu.CompilerParams(use_tc_tiling_on_sc=False),
    scratch_types=[
        pltpu.VMEM((CHUNKS, K), jnp.int32),
        pltpu.VMEM((CHUNKS, K), jnp.int32),
        [pltpu.VMEM((K, HID), jnp.float32)] * NBUF,
        pltpu.VMEM_SHARED((NPAD, HID), jnp.float32),
        [pltpu.SemaphoreType.DMA] * NBUF,
        [pltpu.SemaphoreType.DMA] * NBUF,
    ],
)
def _sc_aggregate(table2_hbm, src_hbm, dst_hbm, zeros_hbm, out_hbm,
                  sidx_v, didx_v, bufs, acc_sh, gsems, ssems):
    cid = lax.axis_index("c")
    sid = lax.axis_index("s")
    table_hbm = table2_hbm.at[cid]
    wid = sid * NC + cid
    r0 = sid * RPT
    n0 = sid * NPT
    pltpu.sync_copy(zeros_hbm.at[pl.ds(r0, RPT)], acc_sh.at[pl.ds(r0, RPT)])
    pltpu.sync_copy(src_hbm.at[wid], sidx_v)
    pltpu.sync_copy(dst_hbm.at[wid], didx_v)
    plsc.subcore_barrier()

    for c in range(GAH):
        pltpu.async_copy(table_hbm.at[sidx_v.at[c]], bufs[c], gsems[c])

    def slot(c, b):
        pltpu.make_async_copy(table_hbm, bufs[b], gsems[b]).wait()
        pltpu.async_copy(bufs[b], acc_sh.at[didx_v.at[c]], ssems[b], add=True)
        b2 = (b + GAH) % NBUF

        @pl.when(c >= GAH)
        def _():
            pltpu.make_async_copy(bufs[b2], acc_sh.at[didx_v.at[0]], ssems[b2]).wait()

        @pl.when(c + GAH < CHUNKS)
        def _():
            pltpu.async_copy(table_hbm.at[sidx_v.at[c + GAH]], bufs[b2], gsems[b2])

    def body(i, carry):
        for b in range(NBUF):
            slot(i * NBUF + b, b)
        return carry

    lax.fori_loop(0, CHUNKS // NBUF, body, 0)
    for c in range(CHUNKS - GAH, CHUNKS):
        b = c % NBUF
        pltpu.make_async_copy(bufs[b], acc_sh.at[didx_v.at[0]], ssems[b]).wait()
    plsc.subcore_barrier()
    pltpu.sync_copy(acc_sh.at[pl.ds(n0, NPT)], out_hbm.at[cid, pl.ds(n0, NPT)])


@functools.partial(
    pl.kernel,
    out_type=jax.ShapeDtypeStruct((E_PAD * HID,), jnp.float32),
    mesh=_mesh,
    compiler_params=pltpu.CompilerParams(
        use_tc_tiling_on_sc=False, needs_layout_passes=False),
    scratch_types=[
        pltpu.VMEM((CHUNKS, K), jnp.int32),
        pltpu.VMEM((CHUNKS, K), jnp.int32),
        pltpu.VMEM((4, 16), jnp.float32),
        [pltpu.VMEM((K, HID), jnp.float32)] * DBUF,
        [pltpu.VMEM((K, HID), jnp.float32)] * DBUF,
        [pltpu.VMEM((K * HID,), jnp.float32)] * DBUF,
        [pltpu.SemaphoreType.DMA] * DBUF,
        [pltpu.SemaphoreType.DMA] * DBUF,
        [pltpu.SemaphoreType.DMA] * DBUF,
    ],
)
def _sc_edge_head(ta2_hbm, tb2_hbm, fsrc_hbm, fdst_hbm, w_hbm, out_hbm,
                  sidx_v, didx_v, w_v, bufa, bufb, outp, gsa, gsb, wsem):
    cid = lax.axis_index("c")
    sid = lax.axis_index("s")
    ta_hbm = ta2_hbm.at[cid]
    tb_hbm = tb2_hbm.at[cid]
    wid = sid * NC + cid
    base = wid * EPW
    pltpu.sync_copy(fsrc_hbm.at[wid], sidx_v)
    pltpu.sync_copy(fdst_hbm.at[wid], didx_v)
    pltpu.sync_copy(w_hbm, w_v)

    LAH = 2
    for c in range(LAH):
        pltpu.async_copy(ta_hbm.at[sidx_v.at[c]], bufa[c], gsa[c])
        pltpu.async_copy(tb_hbm.at[didx_v.at[c]], bufb[c], gsb[c])

    def compute(b):
        ba, bb, po = bufa[b], bufb[b], outp[b]

        def edge4(i, carry):
            for u in range(4):
                e = i * 4 + u
                for q in range(4):
                    po[pl.ds(e * HID + q * 16, 16)] = (
                        ba[e, pl.ds(q * 16, 16)] + bb[e, pl.ds(q * 16, 16)])
            return carry

        lax.fori_loop(0, K // 4, edge4, 0)

    def slot(c, b):
        pltpu.make_async_copy(ta_hbm, bufa[b], gsa[b]).wait()
        pltpu.make_async_copy(tb_hbm, bufb[b], gsb[b]).wait()
        b2 = (b + LAH) % DBUF

        @pl.when(c + LAH < CHUNKS)
        def _():
            pltpu.async_copy(ta_hbm.at[sidx_v.at[c + LAH]], bufa[b2], gsa[b2])
            pltpu.async_copy(tb_hbm.at[didx_v.at[c + LAH]], bufb[b2], gsb[b2])

        @pl.when(c >= DBUF)
        def _():
            pltpu.make_async_copy(outp[b], out_hbm.at[pl.ds(0, K * HID)], wsem[b]).wait()

        compute(b)
        pltpu.async_copy(outp[b], out_hbm.at[pl.ds((base + c * K) * HID, K * HID)],
                         wsem[b])

    def body(i, carry):
        for b in range(DBUF):
            slot(i * DBUF + b, b)
        return carry

    lax.fori_loop(0, CHUNKS // DBUF, body, 0)
    for c in range(CHUNKS - DBUF, CHUNKS):
        b = c % DBUF
        pltpu.make_async_copy(outp[b], out_hbm.at[pl.ds(0, K * HID)], wsem[b]).wait()


def _silu(v):
    return v / (1.0 + jnp.exp(-v))


def _embed_body(x_ref, ts_ref, degp_ref, W_in_ref, b_in_ref, W_t1_ref, b_t1_ref,
                W_t2_ref, b_t2_ref, W_c1_ref, h0_ref, g1s_ref, dinv_ref):
    x = x_ref[...]
    h = jnp.dot(x, W_in_ref[...], preferred_element_type=jnp.float32) + b_in_ref[...]
    t = ts_ref[...].astype(jnp.float32)
    half = HID // 2
    ramp = lax.broadcasted_iota(jnp.int32, (1, half), 1).astype(jnp.float32)
    freqs = jnp.exp(-jnp.log(10000.0) * ramp / (half - 1))
    args = t * freqs
    emb = jnp.concatenate([jnp.sin(args), jnp.cos(args)], axis=-1)
    te = jnp.dot(emb, W_t1_ref[...], preferred_element_type=jnp.float32) + b_t1_ref[...]
    te = _silu(te)
    te = jnp.dot(te, W_t2_ref[...], preferred_element_type=jnp.float32) + b_t2_ref[...]
    h0 = h + te
    deg = degp_ref[0, :, 0:1] + degp_ref[1, :, 0:1] + 1.0
    dinv = lax.rsqrt(deg)
    g1 = jnp.dot(h0, W_c1_ref[...], preferred_element_type=jnp.float32)
    h0_ref[...] = h0
    g1s_ref[...] = g1 * dinv
    dinv_ref[...] = dinv


def _layer_body(aggp_ref, gs_ref, dinv_ref, b_ref, W_next_ref, out_ref):
    dinv = dinv_ref[...]
    z = (aggp_ref[0] + aggp_ref[1] + gs_ref[...]) * dinv + b_ref[...]
    z = jnp.maximum(z, 0.0)
    out_ref[...] = jnp.dot(z, W_next_ref[...], preferred_element_type=jnp.float32) * dinv


def _final_body(aggp_ref, gs_ref, dinv_ref, b_ref, Wfa_ref, bfa_ref, Wfb_ref,
                h1_ref, ta_ref, tb_ref):
    dinv = dinv_ref[...]
    h1 = (aggp_ref[0] + aggp_ref[1] + gs_ref[...]) * dinv + b_ref[...]
    h1_ref[...] = h1
    ta_ref[...] = jnp.dot(h1, Wfa_ref[...], preferred_element_type=jnp.float32) + bfa_ref[...]
    tb_ref[...] = jnp.dot(h1, Wfb_ref[...], preferred_element_type=jnp.float32)


def _decoder_body(s2_ref, w2_ref, bf2_ref, out_ref):
    s = _silu(s2_ref[...])
    out_ref[...] = jnp.dot(s, w2_ref[...], preferred_element_type=jnp.float32) + bf2_ref[...]


def _full(shape):
    return pl.BlockSpec(shape, lambda i: (0,) * len(shape))


def _rows(blk, width):
    return pl.BlockSpec((blk, width), lambda i: (i, 0))


def kernel(x, edge_index, full_edge_index, time_steps,
           W_in, b_in, W_t1, b_t1, W_t2, b_t2,
           W_c1, b_c1, W_c2, b_c2, W_f1, b_f1, W_f2, b_f2):
    f32 = jnp.float32

    def pad_idx(a, fill):
        a = a.astype(jnp.int32)
        a = jnp.concatenate([a, jnp.full((E_PAD - N_EDGES,), fill, jnp.int32)])
        return a.reshape(NW, CHUNKS, K)

    src_p = pad_idx(edge_index[0], 0)
    dst_p = pad_idx(edge_index[1], JUNK)
    fsrc_p = pad_idx(full_edge_index[0], 0)
    fdst_p = pad_idx(full_edge_index[1], 0)

    ts_c = time_steps.astype(jnp.int32).reshape(N_NODES, 1)

    zeros16 = jnp.zeros((NPAD, 16), f32)
    ones16 = jnp.ones((K, 16), f32)
    zeros64 = jnp.zeros((NPAD, HID), f32)

    degp = _sc_degree(dst_p, zeros16, ones16)

    grid_n = N_NODES // NBLK
    h0, g1s, dinv = pl.pallas_call(
        _embed_body,
        grid=(grid_n,),
        in_specs=[
            _rows(NBLK, IN_CH),
            _rows(NBLK, 1),
            pl.BlockSpec((NC, NBLK, 16), lambda i: (0, i, 0)),
            _full((IN_CH, HID)), _full((1, HID)),
            _full((HID, 4 * HID)), _full((1, 4 * HID)),
            _full((4 * HID, HID)), _full((1, HID)),
            _full((HID, HID)),
        ],
        out_specs=[_rows(NBLK, HID), _rows(NBLK, HID), _rows(NBLK, 1)],
        out_shape=[
            jax.ShapeDtypeStruct((N_NODES, HID), f32),
            jax.ShapeDtypeStruct((N_NODES, HID), f32),
            jax.ShapeDtypeStruct((N_NODES, 1), f32),
        ],
    )(x, ts_c, degp, W_in, b_in.reshape(1, HID),
      W_t1, b_t1.reshape(1, 4 * HID), W_t2, b_t2.reshape(1, HID), W_c1)

    g1s2 = jnp.broadcast_to(g1s, (NC, N_NODES, HID))
    agg1 = _sc_aggregate(g1s2, src_p, dst_p, zeros64)

    g2s = pl.pallas_call(
        _layer_body,
        grid=(grid_n,),
        in_specs=[
            pl.BlockSpec((NC, NBLK, HID), lambda i: (0, i, 0)),
            _rows(NBLK, HID), _rows(NBLK, 1), _full((1, HID)), _full((HID, HID)),
        ],
        out_specs=_rows(NBLK, HID),
        out_shape=jax.ShapeDtypeStruct((N_NODES, HID), f32),
    )(agg1, g1s, dinv, b_c1.reshape(1, HID), W_c2)

    g2s2 = jnp.broadcast_to(g2s, (NC, N_NODES, HID))
    agg2 = _sc_aggregate(g2s2, src_p, dst_p, zeros64)

    h1, ta, tb = pl.pallas_call(
        _final_body,
        grid=(grid_n,),
        in_specs=[
            pl.BlockSpec((NC, NBLK, HID), lambda i: (0, i, 0)),
            _rows(NBLK, HID), _rows(NBLK, 1), _full((1, HID)),
            _full((HID, HID)), _full((1, HID)), _full((HID, HID)),
        ],
        out_specs=[_rows(NBLK, HID), _rows(NBLK, HID), _rows(NBLK, HID)],
        out_shape=[
            jax.ShapeDtypeStruct((N_NODES, HID), f32),
            jax.ShapeDtypeStruct((N_NODES, HID), f32),
            jax.ShapeDtypeStruct((N_NODES, HID), f32),
        ],
    )(agg2, g2s, dinv, b_c2.reshape(1, HID),
      W_f1[:HID], b_f1.reshape(1, HID), W_f1[HID:])

    ta2 = jnp.broadcast_to(ta, (NC, N_NODES, HID))
    tb2 = jnp.broadcast_to(tb, (NC, N_NODES, HID))
    s2 = _sc_edge_head(ta2, tb2, fsrc_p, fdst_p, W_f2.reshape(4, 16))
    s2 = s2.reshape(E_PAD // 2, 2 * HID)

    w2st = jnp.zeros((2 * HID, 2), f32)
    w2st = w2st.at[:HID, 0].set(W_f2[:, 0]).at[HID:, 1].set(W_f2[:, 0])
    EB2 = 2000
    grid_e = (N_EDGES // 2) // EB2
    logits2 = pl.pallas_call(
        _decoder_body,
        grid=(grid_e,),
        in_specs=[_rows(EB2, 2 * HID), _full((2 * HID, 2)), _full((1, 2))],
        out_specs=_rows(EB2, 2),
        out_shape=jax.ShapeDtypeStruct((N_EDGES // 2, 2), f32),
    )(s2, w2st, jnp.broadcast_to(b_f2.reshape(1, 1), (1, 2)))

    return (logits2.reshape(N_EDGES, 1), h0, h1)
